# Initial kernel scaffold; baseline (speedup 1.0000x reference)
import jax
import jax.numpy as jnp
from jax.experimental import pallas as pl


def kernel(user_emb, artist_emb, album_emb, item_audio_emb, edge_attr, W1, b1, W2, b2, Wp, bp, edge_src, edge_dst, artist_ids, album_ids):
    raise NotImplementedError("write your pallas kernel here")


# speedup vs baseline: 1.2719x; 1.2719x over previous
import jax
import jax.numpy as jnp
from jax.experimental import pallas as pl
from jax.experimental.pallas import tpu as pltpu

_NU = 25000
_NI = 25000
_NN = _NU + _NI
_NL = 3


def _norm_body(x_ref, o_ref):
    x = x_ref[...]
    n = jnp.sqrt(jnp.sum(x * x, axis=-1, keepdims=True))
    o_ref[...] = x / jnp.maximum(n, 1e-12)


def _pl_normalize(x, blk=1000):
    m, d = x.shape
    return pl.pallas_call(
        _norm_body,
        out_shape=jax.ShapeDtypeStruct((m, d), x.dtype),
        grid=(m // blk,),
        in_specs=[pl.BlockSpec((blk, d), lambda i: (i, 0))],
        out_specs=pl.BlockSpec((blk, d), lambda i: (i, 0)),
    )(x)


def kernel(user_emb, artist_emb, album_emb, item_audio_emb, edge_attr,
           W1, b1, W2, b2, Wp, bp, edge_src, edge_dst, artist_ids, album_ids):
    h = jax.nn.relu(edge_attr @ W1 + b1)
    w = jax.nn.sigmoid(h @ W2 + b2)[:, 0]
    w = jnp.clip(w, 1e-6, None)

    dst_off = edge_dst + _NU
    src_h = jnp.concatenate([edge_src, dst_off])
    dst_h = jnp.concatenate([dst_off, edge_src])
    w_h = jnp.concatenate([w, w])

    deg = jnp.ones((_NN,), jnp.float32)
    deg = deg.at[dst_h].add(w_h)
    dinv = deg ** -0.5
    dinv2 = 1.0 / deg
    norm = dinv[src_h] * w_h * dinv[dst_h]

    user_x = _pl_normalize(user_emb)
    meta = artist_emb[artist_ids] + album_emb[album_ids]
    item_pre = jnp.concatenate([item_audio_emb, meta], axis=-1) @ Wp + bp
    item_x = _pl_normalize(item_pre)
    x = jnp.concatenate([user_x, item_x], axis=0)

    acc = x
    for _ in range(_NL):
        y = jnp.zeros((_NN, 128), jnp.float32).at[dst_h].add(norm[:, None] * x[src_h])
        x = y + dinv2[:, None] * x
        acc = acc + x

    out = _pl_normalize(acc / (_NL + 1))
    return out[:_NU], out[_NU:]


# speedup vs baseline: 1.4757x; 1.1602x over previous
import functools

import jax
import jax.numpy as jnp
from jax import lax
from jax.experimental import pallas as pl
from jax.experimental.pallas import tpu as pltpu
from jax.experimental.pallas import tpu_sc as plsc

_NU = 25000
_NI = 25000
_NN = 50000
_E = 400000
_NL = 3

_BUK = 6272
_NBUK = 8
_NP = _NBUK * _BUK
_NSEG = _NBUK * 16
_CAP = 806912
_MPW = _CAP // 32
_MIDP = 25088


def _norm_body(x_ref, o_ref):
    x = x_ref[...]
    n = jnp.sqrt(jnp.sum(x * x, axis=-1, keepdims=True))
    o_ref[...] = x / jnp.maximum(n, 1e-12)


def _pl_normalize(x, blk):
    m, d = x.shape
    return pl.pallas_call(
        _norm_body,
        out_shape=jax.ShapeDtypeStruct((m, d), x.dtype),
        grid=(m // blk,),
        in_specs=[pl.BlockSpec((blk, d), lambda i: (i, 0))],
        out_specs=pl.BlockSpec((blk, d), lambda i: (i, 0)),
    )(x)


def _tc_edge_mlp(attrT, W1T, b1c, W2T, b2c):
    def body(a_ref, w1_ref, b1_ref, w2_ref, b2_ref, o_ref):
        a = a_ref[...]
        h = jnp.maximum(
            jnp.dot(w1_ref[...], a, preferred_element_type=jnp.float32) + b1_ref[...], 0.0)
        w = jax.nn.sigmoid(
            jnp.dot(w2_ref[...], h, preferred_element_type=jnp.float32) + b2_ref[...])
        o_ref[...] = jnp.maximum(w, 1e-6)

    return pl.pallas_call(
        body,
        out_shape=jax.ShapeDtypeStruct((1, _E), jnp.float32),
        grid=(125,),
        in_specs=[
            pl.BlockSpec((4, 3200), lambda i: (0, i)),
            pl.BlockSpec((32, 4), lambda i: (0, 0)),
            pl.BlockSpec((32, 1), lambda i: (0, 0)),
            pl.BlockSpec((1, 32), lambda i: (0, 0)),
            pl.BlockSpec((1, 1), lambda i: (0, 0)),
        ],
        out_specs=pl.BlockSpec((1, 3200), lambda i: (0, i)),
    )(attrT, W1T, b1c, W2T, b2c)


def _tc_item(audio, meta_p, Wpa, Wpm, bpr):
    def body(au, me, wa, wm, bp_, o):
        pre = (jnp.dot(au[...], wa[...], preferred_element_type=jnp.float32)
               + jnp.dot(me[...], wm[...], preferred_element_type=jnp.float32)
               + bp_[...])
        n = jnp.sqrt(jnp.sum(pre * pre, axis=1, keepdims=True))
        o[...] = pre / jnp.maximum(n, 1e-12)

    return pl.pallas_call(
        body,
        out_shape=jax.ShapeDtypeStruct((_NI, 128), jnp.float32),
        grid=(25,),
        in_specs=[
            pl.BlockSpec((1000, 128), lambda i: (i, 0)),
            pl.BlockSpec((1000, 128), lambda i: (i, 0)),
            pl.BlockSpec((128, 128), lambda i: (0, 0)),
            pl.BlockSpec((128, 128), lambda i: (0, 0)),
            pl.BlockSpec((1, 128), lambda i: (0, 0)),
        ],
        out_specs=pl.BlockSpec((1000, 128), lambda i: (i, 0)),
    )(audio, meta_p, Wpa, Wpm, bpr)


def _sc_mesh():
    return plsc.VectorSubcoreMesh(core_axis_name="c", subcore_axis_name="s")


def _sc_meta(artist_emb, album_emb, aid_p, bid_p):
    @functools.partial(
        pl.kernel,
        out_type=jax.ShapeDtypeStruct((_MIDP, 128), jnp.float32),
        mesh=_sc_mesh(),
        scratch_types=[
            pltpu.VMEM((112,), jnp.int32),
            pltpu.VMEM((112, 128), jnp.float32),
            pltpu.VMEM((112, 128), jnp.float32),
        ],
    )
    def k(art_h, alb_h, aid_h, bid_h, out_h, idx_v, ra, rb):
        c = lax.axis_index("c")
        t = lax.axis_index("s")
        base = (t * 2 + c) * 784

        def chunk(kk, carry):
            r0 = pl.multiple_of(base + kk * 112, 16)
            pltpu.sync_copy(aid_h.at[pl.ds(r0, 112)], idx_v)
            pltpu.sync_copy(art_h.at[idx_v], ra)
            pltpu.sync_copy(bid_h.at[pl.ds(r0, 112)], idx_v)
            pltpu.sync_copy(alb_h.at[idx_v], rb)

            def row(i, cy):
                for d in range(8):
                    sl = pl.ds(d * 16, 16)
                    ra[i, sl] = ra[i, sl] + rb[i, sl]
                return cy

            lax.fori_loop(0, 112, row, 0)
            pltpu.sync_copy(ra, out_h.at[pl.ds(r0, 112)])
            return carry

        lax.fori_loop(0, 7, chunk, 0)

    return k(artist_emb, album_emb, aid_p, bid_p)


def _sc_deg(w_s, dstg_s, meta32):
    @functools.partial(
        pl.kernel,
        out_type=jax.ShapeDtypeStruct((_NP,), jnp.float32),
        mesh=_sc_mesh(),
        scratch_types=[
            pltpu.VMEM((64,), jnp.float32),
            pltpu.VMEM((64,), jnp.int32),
            pltpu.VMEM((64,), jnp.int32),
            pltpu.VMEM((2048,), jnp.int32),
            pltpu.VMEM((1568,), jnp.float32),
            pltpu.VMEM_SHARED((4 * _BUK,), jnp.float32),
        ],
    )
    def k(w_h, dstg_h, m_h, deg_h, wv, dv, dlv, mv, st, dsp):
        c = lax.axis_index("c")
        t = lax.axis_index("s")
        pltpu.sync_copy(m_h, mv)
        shift = c * (4 * _BUK)

        def zrow(i, cy):
            st[pl.ds(i * 16, 16)] = jnp.zeros((16,), jnp.float32)
            return cy

        lax.fori_loop(0, 98, zrow, 0)
        z0 = pl.multiple_of(t * 1568, 32)
        pltpu.sync_copy(st, dsp.at[pl.ds(z0, 1568)])
        plsc.subcore_barrier()

        def bucket(bl, cy):
            mb = mv[pl.ds(pl.multiple_of((bl * 32 + t * 2 + c) * 16, 16), 16)]
            off = mb[0]
            nch = mb[1]

            def chunk(kk, cy2):
                o = pl.multiple_of(off + kk * 64, 64)
                pltpu.sync_copy(w_h.at[pl.ds(o, 64)], wv)
                pltpu.sync_copy(dstg_h.at[pl.ds(o, 64)], dv)
                for q in range(4):
                    sl = pl.ds(q * 16, 16)
                    dlv[sl] = dv[sl] - shift
                pltpu.sync_copy(wv, dsp.at[dlv], add=True)
                return cy2

            lax.fori_loop(0, nch, chunk, 0)
            return cy

        lax.fori_loop(0, 4, bucket, 0)
        plsc.subcore_barrier()
        z1 = pl.multiple_of(t * 1568, 32)
        pltpu.sync_copy(dsp.at[pl.ds(z1, 1568)], st)
        z2 = pl.multiple_of(c * 25088 + t * 1568, 32)
        pltpu.sync_copy(st, deg_h.at[pl.ds(z2, 1568)])

    return k(w_s, dstg_s, meta32)


def _sc_norm(src_s, dstg_s, w_s, dinv):
    @functools.partial(
        pl.kernel,
        out_type=jax.ShapeDtypeStruct((_CAP,), jnp.float32),
        mesh=_sc_mesh(),
        scratch_types=[
            pltpu.VMEM((64,), jnp.int32),
            pltpu.VMEM((64,), jnp.float32),
            pltpu.VMEM((64,), jnp.float32),
            pltpu.VMEM((64,), jnp.float32),
        ],
    )
    def k(src_h, dstg_h, w_h, dinv_h, out_h, iv, av, bv, wv):
        c = lax.axis_index("c")
        t = lax.axis_index("s")
        base = (t * 2 + c) * _MPW

        def chunk(kk, cy):
            o = pl.multiple_of(base + kk * 64, 64)
            pltpu.sync_copy(src_h.at[pl.ds(o, 64)], iv)
            pltpu.sync_copy(dinv_h.at[iv], av)
            pltpu.sync_copy(dstg_h.at[pl.ds(o, 64)], iv)
            pltpu.sync_copy(dinv_h.at[iv], bv)
            pltpu.sync_copy(w_h.at[pl.ds(o, 64)], wv)
            for q in range(4):
                sl = pl.ds(q * 16, 16)
                wv[sl] = wv[sl] * av[sl] * bv[sl]
            pltpu.sync_copy(wv, out_h.at[pl.ds(o, 64)])
            return cy

        lax.fori_loop(0, _MPW // 64, chunk, 0)

    return k(src_s, dstg_s, w_s, dinv)


def _sc_prop(x, accin, dinv2, src_s, dstg_s, norm_s, meta32):
    @functools.partial(
        pl.kernel,
        out_type=(jax.ShapeDtypeStruct((_NP, 128), jnp.float32),
                  jax.ShapeDtypeStruct((_NP, 128), jnp.float32)),
        mesh=_sc_mesh(),
        scratch_types=[
            pltpu.VMEM((2048,), jnp.int32),
            pltpu.VMEM((64,), jnp.int32),
            pltpu.VMEM((64,), jnp.int32),
            pltpu.VMEM((64,), jnp.int32),
            pltpu.VMEM((64,), jnp.float32),
            pltpu.VMEM((64, 128), jnp.float32),
            pltpu.VMEM((64, 128), jnp.float32),
            pltpu.VMEM((64, 128), jnp.float32),
            pltpu.VMEM((64, 128), jnp.float32),
            pltpu.VMEM((64,), jnp.float32),
            pltpu.VMEM_SHARED((_BUK, 128), jnp.float32),
        ],
    )
    def k(x_h, ai_h, d2_h, src_h, dstg_h, nrm_h, m_h, y_h, ao_h,
          mv, iv, dv, dlv, nv, rows, sb, xb, ab, dvv, asp):
        c = lax.axis_index("c")
        t = lax.axis_index("s")
        pltpu.sync_copy(m_h, mv)

        def bucket(bl, cy):
            gb = c * 4 + bl
            nbase = gb * _BUK
            mb = mv[pl.ds(pl.multiple_of((bl * 32 + t * 2 + c) * 16, 16), 16)]
            off = mb[0]
            nch = mb[1]

            def zr(i, cy2):
                for d in range(8):
                    sb[i, pl.ds(d * 16, 16)] = jnp.zeros((16,), jnp.float32)
                return cy2

            lax.fori_loop(0, 64, zr, 0)

            def zc(i, cy2):
                pltpu.sync_copy(sb, asp.at[pl.ds(t * 392 + i * 64, 64)])
                return cy2

            lax.fori_loop(0, 6, zc, 0)
            pltpu.sync_copy(sb.at[pl.ds(0, 8)], asp.at[pl.ds(t * 392 + 384, 8)])
            plsc.subcore_barrier()

            def ech(kk, cy2):
                o = pl.multiple_of(off + kk * 64, 64)
                pltpu.sync_copy(src_h.at[pl.ds(o, 64)], iv)
                pltpu.sync_copy(dstg_h.at[pl.ds(o, 64)], dv)
                pltpu.sync_copy(nrm_h.at[pl.ds(o, 64)], nv)
                pltpu.sync_copy(x_h.at[iv], rows)

                def sgrp(q, cy3):
                    s16 = nv[pl.ds(q * 16, 16)]
                    for l in range(16):
                        i = q * 16 + l
                        s = s16[l]
                        for d in range(8):
                            sl = pl.ds(d * 16, 16)
                            rows[i, sl] = rows[i, sl] * s
                    return cy3

                lax.fori_loop(0, 4, sgrp, 0)
                for q in range(4):
                    sl = pl.ds(q * 16, 16)
                    dlv[sl] = dv[sl] - nbase
                pltpu.sync_copy(rows, asp.at[dlv], add=True)
                return cy2

            lax.fori_loop(0, nch, ech, 0)
            plsc.subcore_barrier()

            nout = jnp.where(t < 2, 7, 6)

            def och(kk, cy2):
                j = t + kk * 16
                g0 = pl.multiple_of(nbase + j * 64, 64)
                pltpu.sync_copy(asp.at[pl.ds(j * 64, 64)], sb)
                pltpu.sync_copy(x_h.at[pl.ds(g0, 64)], xb)
                pltpu.sync_copy(ai_h.at[pl.ds(g0, 64)], ab)
                pltpu.sync_copy(d2_h.at[pl.ds(g0, 64)], dvv)

                def ogrp(q, cy3):
                    s16 = dvv[pl.ds(q * 16, 16)]
                    for l in range(16):
                        i = q * 16 + l
                        s = s16[l]
                        for d in range(8):
                            sl = pl.ds(d * 16, 16)
                            y = sb[i, sl] + xb[i, sl] * s
                            sb[i, sl] = y
                            ab[i, sl] = ab[i, sl] + y
                    return cy3

                lax.fori_loop(0, 4, ogrp, 0)
                pltpu.sync_copy(sb, y_h.at[pl.ds(g0, 64)])
                pltpu.sync_copy(ab, ao_h.at[pl.ds(g0, 64)])
                return cy2

            lax.fori_loop(0, nout, och, 0)
            plsc.subcore_barrier()
            return cy

        lax.fori_loop(0, 4, bucket, 0)

    return k(x, accin, dinv2, src_s, dstg_s, norm_s, meta32)


def kernel(user_emb, artist_emb, album_emb, item_audio_emb, edge_attr,
           W1, b1, W2, b2, Wp, bp, edge_src, edge_dst, artist_ids, album_ids):
    f32 = jnp.float32

    wrow = _tc_edge_mlp(edge_attr.T, W1.T, b1.reshape(32, 1), W2.T, b2.reshape(1, 1))
    w = wrow[0]

    dst_off = edge_dst + _NU
    src_h = jnp.concatenate([edge_src, dst_off])
    dst_h = jnp.concatenate([dst_off, edge_src])
    w_h = jnp.concatenate([w, w])
    key = dst_h // _BUK
    onehot = (key[:, None] == jnp.arange(_NBUK, dtype=key.dtype)[None, :]).astype(jnp.int32)
    ranks = jnp.cumsum(onehot, axis=0)
    rank = jnp.take_along_axis(ranks, key[:, None], axis=1)[:, 0] - 1
    cnt = ranks[-1]
    per_tile = (cnt + 15) // 16
    pt_e = per_tile[key]
    t_idx = rank // pt_e
    r_in_t = rank - t_idx * pt_e
    seg_cnt = jnp.clip(cnt[:, None] - jnp.arange(16, dtype=jnp.int32)[None, :] * per_tile[:, None],
                       0, per_tile[:, None])
    scf = (((seg_cnt + 63) // 64) * 64).reshape(-1).astype(jnp.int32)
    seg_off = jnp.concatenate([jnp.zeros((1,), jnp.int32),
                               jnp.cumsum(scf)[:-1].astype(jnp.int32)])
    segnch = scf // 64
    pos = seg_off[key * 16 + t_idx] + r_in_t
    src_s = jnp.zeros((_CAP,), jnp.int32).at[pos].set(src_h)
    bucket_start = seg_off[:: 16]
    b_of_p = jnp.searchsorted(bucket_start, jnp.arange(_CAP, dtype=jnp.int32),
                              side="right").astype(jnp.int32) - 1
    b_of_p = jnp.clip(b_of_p, 0, _NBUK - 1)
    dstg_s = (b_of_p * _BUK).at[pos].set(dst_h)
    w_s = jnp.zeros((_CAP,), f32).at[pos].set(w_h)

    tw = jnp.arange(32, dtype=jnp.int32) // 2
    cw = jnp.arange(32, dtype=jnp.int32) % 2
    rows_m = []
    for bl in range(4):
        segb = (cw * 4 + bl) * 16 + tw
        rows_m.append(jnp.stack([seg_off[segb], segnch[segb]] +
                                [jnp.zeros((32,), jnp.int32)] * 14, axis=1))
    meta32 = jnp.concatenate(rows_m, axis=0).reshape(-1)

    degs = _sc_deg(w_s, dstg_s, meta32)
    deg = degs + 1.0
    dinv = deg ** -0.5
    dinv2 = 1.0 / deg

    norm_s = _sc_norm(src_s, dstg_s, w_s, dinv)

    aid_p = jnp.zeros((_MIDP,), jnp.int32).at[:_NI].set(artist_ids)
    bid_p = jnp.zeros((_MIDP,), jnp.int32).at[:_NI].set(album_ids)
    meta_p = _sc_meta(artist_emb, album_emb, aid_p, bid_p)

    user_x = _pl_normalize(user_emb, 1000)
    item_x = _tc_item(item_audio_emb, meta_p[:_NI], Wp[:128], Wp[128:], bp.reshape(1, 128))
    x = jnp.concatenate([user_x, item_x, jnp.zeros((_NP - _NN, 128), f32)], axis=0)

    acc = x
    for _ in range(_NL):
        x, acc = _sc_prop(x, acc, dinv2, src_s, dstg_s, norm_s, meta32)

    out = _pl_normalize(acc * 0.25, 784)
    return out[:_NU], out[_NU:_NN]


# speedup vs baseline: 1.8474x; 1.2519x over previous
import functools

import jax
import jax.numpy as jnp
from jax import lax
from jax.experimental import pallas as pl
from jax.experimental.pallas import tpu as pltpu
from jax.experimental.pallas import tpu_sc as plsc

_NU = 25000
_NI = 25000
_NN = 50000
_E = 400000
_E2 = 800000
_NL = 3

_BUK = 6272
_NBUK = 8
_NP = _NBUK * _BUK
_NSEG = _NBUK * 16
_CH = 128
_CAP = _E2 + _NSEG * (_CH - 1)
_TRASH = _CAP
_CAPX = _CAP + _CH
_E2P = 819200
_MIDP = 25088


def _norm_body(x_ref, o_ref):
    x = x_ref[...]
    n = jnp.sqrt(jnp.sum(x * x, axis=-1, keepdims=True))
    o_ref[...] = x / jnp.maximum(n, 1e-12)


def _pl_normalize(x, blk):
    m, d = x.shape
    return pl.pallas_call(
        _norm_body,
        out_shape=jax.ShapeDtypeStruct((m, d), x.dtype),
        grid=(m // blk,),
        in_specs=[pl.BlockSpec((blk, d), lambda i: (i, 0))],
        out_specs=pl.BlockSpec((blk, d), lambda i: (i, 0)),
    )(x)


def _tc_edge_mlp(attrT, W1T, b1c, W2T, b2c):
    def body(a_ref, w1_ref, b1_ref, w2_ref, b2_ref, o_ref):
        a = a_ref[...]
        h = jnp.maximum(
            jnp.dot(w1_ref[...], a, preferred_element_type=jnp.float32) + b1_ref[...], 0.0)
        w = jax.nn.sigmoid(
            jnp.dot(w2_ref[...], h, preferred_element_type=jnp.float32) + b2_ref[...])
        o_ref[...] = jnp.maximum(w, 1e-6)

    return pl.pallas_call(
        body,
        out_shape=jax.ShapeDtypeStruct((1, _E), jnp.float32),
        grid=(125,),
        in_specs=[
            pl.BlockSpec((4, 3200), lambda i: (0, i)),
            pl.BlockSpec((32, 4), lambda i: (0, 0)),
            pl.BlockSpec((32, 1), lambda i: (0, 0)),
            pl.BlockSpec((1, 32), lambda i: (0, 0)),
            pl.BlockSpec((1, 1), lambda i: (0, 0)),
        ],
        out_specs=pl.BlockSpec((1, 3200), lambda i: (0, i)),
    )(attrT, W1T, b1c, W2T, b2c)


def _tc_item(audio, meta_p, Wpa, Wpm, bpr):
    def body(au, me, wa, wm, bp_, o):
        pre = (jnp.dot(au[...], wa[...], preferred_element_type=jnp.float32)
               + jnp.dot(me[...], wm[...], preferred_element_type=jnp.float32)
               + bp_[...])
        n = jnp.sqrt(jnp.sum(pre * pre, axis=1, keepdims=True))
        o[...] = pre / jnp.maximum(n, 1e-12)

    return pl.pallas_call(
        body,
        out_shape=jax.ShapeDtypeStruct((_NI, 128), jnp.float32),
        grid=(25,),
        in_specs=[
            pl.BlockSpec((1000, 128), lambda i: (i, 0)),
            pl.BlockSpec((1000, 128), lambda i: (i, 0)),
            pl.BlockSpec((128, 128), lambda i: (0, 0)),
            pl.BlockSpec((128, 128), lambda i: (0, 0)),
            pl.BlockSpec((1, 128), lambda i: (0, 0)),
        ],
        out_specs=pl.BlockSpec((1000, 128), lambda i: (i, 0)),
    )(audio, meta_p, Wpa, Wpm, bpr)


def _sc_mesh():
    return plsc.VectorSubcoreMesh(core_axis_name="c", subcore_axis_name="s")


def _sc_meta(artist_emb, album_emb, aid_p, bid_p):
    @functools.partial(
        pl.kernel,
        out_type=jax.ShapeDtypeStruct((_MIDP, 128), jnp.float32),
        mesh=_sc_mesh(),
        scratch_types=[
            pltpu.VMEM((112,), jnp.int32),
            pltpu.VMEM((112, 128), jnp.float32),
            pltpu.VMEM((112, 128), jnp.float32),
        ],
    )
    def k(art_h, alb_h, aid_h, bid_h, out_h, idx_v, ra, rb):
        c = lax.axis_index("c")
        t = lax.axis_index("s")
        base = (t * 2 + c) * 784

        def chunk(kk, carry):
            r0 = pl.multiple_of(base + kk * 112, 16)
            pltpu.sync_copy(aid_h.at[pl.ds(r0, 112)], idx_v)
            pltpu.sync_copy(art_h.at[idx_v], ra)
            pltpu.sync_copy(bid_h.at[pl.ds(r0, 112)], idx_v)
            pltpu.sync_copy(alb_h.at[idx_v], rb)

            def row(i, cy):
                for d in range(8):
                    sl = pl.ds(d * 16, 16)
                    ra[i, sl] = ra[i, sl] + rb[i, sl]
                return cy

            lax.fori_loop(0, 112, row, 0)
            pltpu.sync_copy(ra, out_h.at[pl.ds(r0, 112)])
            return carry

        lax.fori_loop(0, 7, chunk, 0)

    return k(artist_emb, album_emb, aid_p, bid_p)


def _sc_deg(pkd_s, w_s, meta32):
    @functools.partial(
        pl.kernel,
        out_type=jax.ShapeDtypeStruct((_NP,), jnp.float32),
        mesh=_sc_mesh(),
        scratch_types=[
            pltpu.VMEM((2048,), jnp.int32),
            pltpu.VMEM((128,), jnp.int32),
            pltpu.VMEM((128,), jnp.int32),
            pltpu.VMEM((128,), jnp.float32),
            pltpu.VMEM((3136,), jnp.float32),
            pltpu.VMEM_SHARED((4 * _BUK,), jnp.float32),
        ],
    )
    def k(pkd_h, w_h, m_h, deg_h, mv, pv, dlv, wv, st, dsp):
        c = lax.axis_index("c")
        t = lax.axis_index("s")
        pltpu.sync_copy(m_h, mv)

        def zrow(i, cy):
            st[pl.ds(i * 16, 16)] = jnp.zeros((16,), jnp.float32)
            return cy

        lax.fori_loop(0, 196, zrow, 0)
        z0 = pl.multiple_of(t * 1568, 32)
        pltpu.sync_copy(st.at[pl.ds(0, 1568)], dsp.at[pl.ds(z0, 1568)])
        plsc.subcore_barrier()

        def bucket(bl, cy):
            mb = mv[pl.ds(pl.multiple_of((bl * 32 + t * 2 + c) * 16, 16), 16)]
            off = mb[0]
            nch = mb[1]
            sh = bl * _BUK

            def chunk(kk, cy2):
                o = pl.multiple_of(off + kk * _CH, _CH)
                pltpu.sync_copy(pkd_h.at[pl.ds(o, _CH)], pv)
                pltpu.sync_copy(w_h.at[pl.ds(o, _CH)], wv)
                for q in range(8):
                    sl = pl.ds(q * 16, 16)
                    dlv[sl] = (pv[sl] & 8191) + sh
                pltpu.sync_copy(wv, dsp.at[dlv], add=True)
                return cy2

            lax.fori_loop(0, nch, chunk, 0)
            return cy

        lax.fori_loop(0, 4, bucket, 0)
        plsc.subcore_barrier()
        z1 = pl.multiple_of(t * 1568, 32)
        pltpu.sync_copy(dsp.at[pl.ds(z1, 1568)], st.at[pl.ds(0, 1568)])
        z2 = pl.multiple_of(c * (4 * _BUK) + t * 1568, 32)
        pltpu.sync_copy(st.at[pl.ds(0, 1568)], deg_h.at[pl.ds(z2, 1568)])

    return k(pkd_s, w_s, meta32)


def _sc_norm(pkd_s, w_s, dinv, meta32):
    @functools.partial(
        pl.kernel,
        out_type=jax.ShapeDtypeStruct((_CAPX,), jnp.float32),
        mesh=_sc_mesh(),
        scratch_types=[
            pltpu.VMEM((2048,), jnp.int32),
            pltpu.VMEM((128,), jnp.int32),
            pltpu.VMEM((128,), jnp.int32),
            pltpu.VMEM((128,), jnp.float32),
            pltpu.VMEM((128,), jnp.float32),
            pltpu.VMEM((128,), jnp.float32),
        ],
    )
    def k(pkd_h, w_h, dinv_h, m_h, out_h, mv, pv, iv, av, bv, wv):
        c = lax.axis_index("c")
        t = lax.axis_index("s")
        pltpu.sync_copy(m_h, mv)

        def bucket(bl, cy):
            gb = c * 4 + bl
            nbase = gb * _BUK
            mb = mv[pl.ds(pl.multiple_of((bl * 32 + t * 2 + c) * 16, 16), 16)]
            off = mb[0]
            nch = mb[1]

            def chunk(kk, cy2):
                o = pl.multiple_of(off + kk * _CH, _CH)
                pltpu.sync_copy(pkd_h.at[pl.ds(o, _CH)], pv)
                for q in range(8):
                    sl = pl.ds(q * 16, 16)
                    iv[sl] = lax.shift_right_logical(pv[sl], 13)
                pltpu.sync_copy(dinv_h.at[iv], av)
                for q in range(8):
                    sl = pl.ds(q * 16, 16)
                    iv[sl] = (pv[sl] & 8191) + nbase
                pltpu.sync_copy(dinv_h.at[iv], bv)
                pltpu.sync_copy(w_h.at[pl.ds(o, _CH)], wv)
                for q in range(8):
                    sl = pl.ds(q * 16, 16)
                    wv[sl] = wv[sl] * av[sl] * bv[sl]
                pltpu.sync_copy(wv, out_h.at[pl.ds(o, _CH)])
                return cy2

            lax.fori_loop(0, nch, chunk, 0)
            return cy

        lax.fori_loop(0, 4, bucket, 0)

    return k(pkd_s, w_s, dinv, meta32)


def _sc_prop(x, accin, dinv2, pkd_s, norm_s, meta32):
    @functools.partial(
        pl.kernel,
        out_type=(jax.ShapeDtypeStruct((_NP, 128), jnp.float32),
                  jax.ShapeDtypeStruct((_NP, 128), jnp.float32)),
        mesh=_sc_mesh(),
        scratch_types=[
            pltpu.VMEM((2048,), jnp.int32),
            pltpu.VMEM((128,), jnp.int32),
            pltpu.VMEM((128,), jnp.int32),
            pltpu.VMEM((128,), jnp.float32),
            pltpu.VMEM((128,), jnp.float32),
            pltpu.VMEM((128,), jnp.int32),
            pltpu.VMEM((128,), jnp.int32),
            pltpu.VMEM((128,), jnp.int32),
            pltpu.VMEM((128,), jnp.int32),
            pltpu.VMEM((128, 128), jnp.float32),
            pltpu.VMEM((128, 128), jnp.float32),
            pltpu.VMEM((64, 128), jnp.float32),
            pltpu.VMEM((64, 128), jnp.float32),
            pltpu.VMEM((64, 128), jnp.float32),
            pltpu.VMEM((64,), jnp.float32),
            pltpu.VMEM_SHARED((_BUK, 128), jnp.float32),
            pltpu.SemaphoreType.DMA,
            pltpu.SemaphoreType.DMA,
            pltpu.SemaphoreType.DMA,
            pltpu.SemaphoreType.DMA,
        ],
    )
    def k(x_h, ai_h, d2_h, pkd_h, nrm_h, m_h, y_h, ao_h,
          mv, pk0, pk1, nv0, nv1, iv0, iv1, dl0, dl1, rw0, rw1,
          sb, xb, ab, dv64, asp, semG0, semG1, semS0, semS1):
        c = lax.axis_index("c")
        t = lax.axis_index("s")
        pltpu.sync_copy(m_h, mv)
        sets = ((pk0, nv0, iv0, dl0, rw0, semG0, semS0),
                (pk1, nv1, iv1, dl1, rw1, semG1, semS1))

        def load_and_fire(kk, off, s):
            pk, nv, iv, dl, rw, semG, _ = sets[s]
            o = pl.multiple_of(off + kk * _CH, _CH)
            pltpu.sync_copy(pkd_h.at[pl.ds(o, _CH)], pk)
            pltpu.sync_copy(nrm_h.at[pl.ds(o, _CH)], nv)
            for q in range(8):
                sl = pl.ds(q * 16, 16)
                p = pk[sl]
                iv[sl] = lax.shift_right_logical(p, 13)
                dl[sl] = p & 8191
            pltpu.async_copy(x_h.at[iv], rw, semG)

        def scale_and_scatter(s):
            _, nv, _, dl, rw, _, semS = sets[s]

            def sgrp(q, cy3):
                s16 = nv[pl.ds(q * 16, 16)]
                for l in range(16):
                    i = q * 16 + l
                    sc_ = s16[l]
                    for d in range(8):
                        sl = pl.ds(d * 16, 16)
                        rw[i, sl] = rw[i, sl] * sc_
                return cy3

            lax.fori_loop(0, 8, sgrp, 0)
            pltpu.async_copy(rw, asp.at[dl], semS, add=True)

        def wait_gather(s):
            _, _, iv, _, rw, semG, _ = sets[s]
            pltpu.make_async_copy(x_h.at[iv], rw, semG).wait()

        def wait_scatter(s):
            _, _, _, dl, rw, _, semS = sets[s]
            pltpu.make_async_copy(rw, asp.at[dl], semS).wait()

        def bucket(bl, cy):
            gb = c * 4 + bl
            nbase = gb * _BUK
            mb = mv[pl.ds(pl.multiple_of((bl * 32 + t * 2 + c) * 16, 16), 16)]
            off = mb[0]
            nch = mb[1]

            def zr(i, cy2):
                for d in range(8):
                    sb[i, pl.ds(d * 16, 16)] = jnp.zeros((16,), jnp.float32)
                return cy2

            lax.fori_loop(0, 64, zr, 0)

            def zc(i, cy2):
                pltpu.sync_copy(sb, asp.at[pl.ds(t * 392 + i * 64, 64)])
                return cy2

            lax.fori_loop(0, 6, zc, 0)
            pltpu.sync_copy(sb.at[pl.ds(0, 8)], asp.at[pl.ds(t * 392 + 384, 8)])
            plsc.subcore_barrier()

            def step(kk, cy2):
                o = pl.multiple_of(off + kk * _CH, _CH)
                pltpu.sync_copy(pkd_h.at[pl.ds(o, _CH)], pk0)
                pltpu.sync_copy(nrm_h.at[pl.ds(o, _CH)], nv0)
                for q in range(8):
                    sl = pl.ds(q * 16, 16)
                    p = pk0[sl]
                    iv0[sl] = lax.shift_right_logical(p, 13)
                    dl0[sl] = p & 8191
                pltpu.sync_copy(x_h.at[iv0], rw0)

                def sgrp(q, cy3):
                    s16 = nv0[pl.ds(q * 16, 16)]
                    for l in range(16):
                        i = q * 16 + l
                        sc_ = s16[l]
                        for d in range(8):
                            sl = pl.ds(d * 16, 16)
                            rw0[i, sl] = rw0[i, sl] * sc_
                    return cy3

                lax.fori_loop(0, 8, sgrp, 0)
                pltpu.sync_copy(rw0, asp.at[dl0], add=True)
                return cy2

            lax.fori_loop(0, nch, step, 0)
            plsc.subcore_barrier()

            nout = jnp.where(t < 2, 7, 6)

            def och(kk, cy2):
                j = t + kk * 16
                g0 = pl.multiple_of(nbase + j * 64, 64)
                pltpu.sync_copy(asp.at[pl.ds(j * 64, 64)], sb)
                pltpu.sync_copy(x_h.at[pl.ds(g0, 64)], xb)
                pltpu.sync_copy(ai_h.at[pl.ds(g0, 64)], ab)
                pltpu.sync_copy(d2_h.at[pl.ds(g0, 64)], dv64)

                def ogrp(q, cy3):
                    s16 = dv64[pl.ds(q * 16, 16)]
                    for l in range(16):
                        i = q * 16 + l
                        sc_ = s16[l]
                        for d in range(8):
                            sl = pl.ds(d * 16, 16)
                            y = sb[i, sl] + xb[i, sl] * sc_
                            sb[i, sl] = y
                            ab[i, sl] = ab[i, sl] + y
                    return cy3

                lax.fori_loop(0, 4, ogrp, 0)
                pltpu.sync_copy(sb, y_h.at[pl.ds(g0, 64)])
                pltpu.sync_copy(ab, ao_h.at[pl.ds(g0, 64)])
                return cy2

            lax.fori_loop(0, nout, och, 0)
            plsc.subcore_barrier()
            return cy

        lax.fori_loop(0, 4, bucket, 0)

    return k(x, accin, dinv2, pkd_s, norm_s, meta32)


def kernel(user_emb, artist_emb, album_emb, item_audio_emb, edge_attr,
           W1, b1, W2, b2, Wp, bp, edge_src, edge_dst, artist_ids, album_ids):
    f32 = jnp.float32
    i32 = jnp.int32

    wrow = _tc_edge_mlp(edge_attr.T, W1.T, b1.reshape(32, 1), W2.T, b2.reshape(1, 1))
    w = wrow[0]

    dst_off = edge_dst + _NU
    src_h = jnp.concatenate([edge_src, dst_off])
    dst_h = jnp.concatenate([dst_off, edge_src])
    w_h = jnp.concatenate([w, w])
    key = dst_h // _BUK
    onehot = (key[:, None] == jnp.arange(_NBUK, dtype=key.dtype)[None, :]).astype(i32)
    ranks = jnp.cumsum(onehot, axis=0)
    rank = jnp.take_along_axis(ranks, key[:, None], axis=1)[:, 0] - 1
    cnt = ranks[-1]
    per_tile = (cnt + 15) // 16
    pt_e = per_tile[key]
    t_idx = rank // pt_e
    r_in_t = rank - t_idx * pt_e
    seg_cnt = jnp.clip(cnt[:, None] - jnp.arange(16, dtype=i32)[None, :] * per_tile[:, None],
                       0, per_tile[:, None])
    cnt_f = seg_cnt.reshape(-1).astype(i32)
    scf = ((cnt_f + _CH - 1) // _CH) * _CH
    seg_off = jnp.concatenate([jnp.zeros((1,), i32), jnp.cumsum(scf)[:-1].astype(i32)])
    segnch = scf // _CH
    pos = seg_off[key * 16 + t_idx] + r_in_t
    pkd = src_h * 8192 + (dst_h - key * _BUK)
    pkd_s = jnp.zeros((_CAPX,), i32).at[pos].set(
        pkd, unique_indices=True, mode="promise_in_bounds")
    w_s = jnp.zeros((_CAPX,), f32).at[pos].set(
        w_h, unique_indices=True, mode="promise_in_bounds")

    tw = jnp.arange(32, dtype=i32) // 2
    cw = jnp.arange(32, dtype=i32) % 2
    rows_m = []
    for bl in range(4):
        segb = (cw * 4 + bl) * 16 + tw
        rows_m.append(jnp.stack([seg_off[segb], segnch[segb]] +
                                [jnp.zeros((32,), i32)] * 14, axis=1))
    meta32 = jnp.concatenate(rows_m, axis=0).reshape(-1)

    degs = _sc_deg(pkd_s, w_s, meta32)
    deg = degs + 1.0
    dinv = deg ** -0.5
    dinv2 = 1.0 / deg

    norm_s = _sc_norm(pkd_s, w_s, dinv, meta32)

    aid_p = jnp.zeros((_MIDP,), i32).at[:_NI].set(artist_ids)
    bid_p = jnp.zeros((_MIDP,), i32).at[:_NI].set(album_ids)
    meta_p = _sc_meta(artist_emb, album_emb, aid_p, bid_p)

    user_x = _pl_normalize(user_emb, 1000)
    item_x = _tc_item(item_audio_emb, meta_p[:_NI], Wp[:128], Wp[128:], bp.reshape(1, 128))
    x = jnp.concatenate([user_x, item_x, jnp.zeros((_NP - _NN, 128), f32)], axis=0)

    acc = x
    for _ in range(_NL):
        x, acc = _sc_prop(x, acc, dinv2, pkd_s, norm_s, meta32)

    out = _pl_normalize(acc * 0.25, 784)
    return out[:_NU], out[_NU:_NN]


# speedup vs baseline: 2.8498x; 1.5426x over previous
import functools

import jax
import jax.numpy as jnp
from jax import lax
from jax.experimental import pallas as pl
from jax.experimental.pallas import tpu as pltpu
from jax.experimental.pallas import tpu_sc as plsc

_NU = 25000
_NI = 25000
_NN = 50000
_E = 400000
_E2 = 800000
_NL = 3

_BUK = 6272
_NBUK = 8
_NP = _NBUK * _BUK
_NSEG = _NBUK * 16
_CH = 128
_NUP = 25088
_HCAP = 408576
_HS = 409600
_CAPX = 2 * _HCAP
_EHS = 401408
_MIDP = 25088


def _norm_body(x_ref, o_ref):
    x = x_ref[...]
    n = jnp.sqrt(jnp.sum(x * x, axis=-1, keepdims=True))
    o_ref[...] = x / jnp.maximum(n, 1e-12)


def _pl_normalize(x, blk):
    m, d = x.shape
    return pl.pallas_call(
        _norm_body,
        out_shape=jax.ShapeDtypeStruct((m, d), x.dtype),
        grid=(m // blk,),
        in_specs=[pl.BlockSpec((blk, d), lambda i: (i, 0))],
        out_specs=pl.BlockSpec((blk, d), lambda i: (i, 0)),
    )(x)


def _tc_edge_mlp(attrT, W1T, b1c, W2T, b2c):
    def body(a_ref, w1_ref, b1_ref, w2_ref, b2_ref, o_ref):
        a = a_ref[...]
        h = jnp.maximum(
            jnp.dot(w1_ref[...], a, preferred_element_type=jnp.float32) + b1_ref[...], 0.0)
        w = jax.nn.sigmoid(
            jnp.dot(w2_ref[...], h, preferred_element_type=jnp.float32) + b2_ref[...])
        o_ref[...] = jnp.maximum(w, 1e-6)

    return pl.pallas_call(
        body,
        out_shape=jax.ShapeDtypeStruct((1, _E), jnp.float32),
        grid=(125,),
        in_specs=[
            pl.BlockSpec((4, 3200), lambda i: (0, i)),
            pl.BlockSpec((32, 4), lambda i: (0, 0)),
            pl.BlockSpec((32, 1), lambda i: (0, 0)),
            pl.BlockSpec((1, 32), lambda i: (0, 0)),
            pl.BlockSpec((1, 1), lambda i: (0, 0)),
        ],
        out_specs=pl.BlockSpec((1, 3200), lambda i: (0, i)),
    )(attrT, W1T, b1c, W2T, b2c)


def _tc_item(audio, meta_p, Wpa, Wpm, bpr):
    def body(au, me, wa, wm, bp_, o):
        pre = (jnp.dot(au[...], wa[...], preferred_element_type=jnp.float32)
               + jnp.dot(me[...], wm[...], preferred_element_type=jnp.float32)
               + bp_[...])
        n = jnp.sqrt(jnp.sum(pre * pre, axis=1, keepdims=True))
        o[...] = pre / jnp.maximum(n, 1e-12)

    return pl.pallas_call(
        body,
        out_shape=jax.ShapeDtypeStruct((_NI, 128), jnp.float32),
        grid=(25,),
        in_specs=[
            pl.BlockSpec((1000, 128), lambda i: (i, 0)),
            pl.BlockSpec((1000, 128), lambda i: (i, 0)),
            pl.BlockSpec((128, 128), lambda i: (0, 0)),
            pl.BlockSpec((128, 128), lambda i: (0, 0)),
            pl.BlockSpec((1, 128), lambda i: (0, 0)),
        ],
        out_specs=pl.BlockSpec((1000, 128), lambda i: (i, 0)),
    )(audio, meta_p, Wpa, Wpm, bpr)


def _sc_mesh():
    return plsc.VectorSubcoreMesh(core_axis_name="c", subcore_axis_name="s")


def _sc_meta(artist_emb, album_emb, aid_p, bid_p):
    @functools.partial(
        pl.kernel,
        out_type=jax.ShapeDtypeStruct((_MIDP, 128), jnp.float32),
        mesh=_sc_mesh(),
        scratch_types=[
            pltpu.VMEM((112,), jnp.int32),
            pltpu.VMEM((112, 128), jnp.float32),
            pltpu.VMEM((112, 128), jnp.float32),
        ],
    )
    def k(art_h, alb_h, aid_h, bid_h, out_h, idx_v, ra, rb):
        c = lax.axis_index("c")
        t = lax.axis_index("s")
        base = (t * 2 + c) * 784

        def chunk(kk, carry):
            r0 = pl.multiple_of(base + kk * 112, 16)
            pltpu.sync_copy(aid_h.at[pl.ds(r0, 112)], idx_v)
            pltpu.sync_copy(art_h.at[idx_v], ra)
            pltpu.sync_copy(bid_h.at[pl.ds(r0, 112)], idx_v)
            pltpu.sync_copy(alb_h.at[idx_v], rb)

            def row(i, cy):
                for d in range(8):
                    sl = pl.ds(d * 16, 16)
                    ra[i, sl] = ra[i, sl] + rb[i, sl]
                return cy

            lax.fori_loop(0, 112, row, 0)
            pltpu.sync_copy(ra, out_h.at[pl.ds(r0, 112)])
            return carry

        lax.fori_loop(0, 7, chunk, 0)

    return k(artist_emb, album_emb, aid_p, bid_p)


def _sc_build(pos_f, pkd_f, dloc_f, w_f):
    @functools.partial(
        pl.kernel,
        out_type=(jax.ShapeDtypeStruct((_CAPX,), jnp.int32),
                  jax.ShapeDtypeStruct((_CAPX,), jnp.float32),
                  jax.ShapeDtypeStruct((2 * _NUP,), jnp.float32)),
        mesh=_sc_mesh(),
        scratch_types=[
            pltpu.VMEM((512,), jnp.int32),
            pltpu.VMEM((512,), jnp.float32),
            pltpu.VMEM((128,), jnp.int32),
            pltpu.VMEM((128,), jnp.int32),
            pltpu.VMEM((128,), jnp.int32),
            pltpu.VMEM((128,), jnp.int32),
            pltpu.VMEM((128,), jnp.int32),
            pltpu.VMEM((128,), jnp.int32),
            pltpu.VMEM((128,), jnp.int32),
            pltpu.VMEM((128,), jnp.int32),
            pltpu.VMEM((3200,), jnp.int32),
            pltpu.VMEM((3200,), jnp.float32),
            pltpu.VMEM_SHARED((_HS,), jnp.int32),
            pltpu.VMEM_SHARED((_HS,), jnp.float32),
            pltpu.VMEM_SHARED((_NUP,), jnp.float32),
            pltpu.SemaphoreType.DMA,
            pltpu.SemaphoreType.DMA,
        ],
    )
    def k(pos_h, pkd_h, dloc_h, w_h, opkd_h, ow_h, odeg_h,
          vv, wv, p0, p1, p2, p3, d0, d1, d2, d3, zi, zf,
          psp, wsp, dsp, semL, semS):
        c = lax.axis_index("c")
        t = lax.axis_index("s")
        ebase = c * _EHS + t * 25088
        pbufs = [p0, p1, p2, p3]
        dbufs = [d0, d1, d2, d3]

        def zr(i, cy):
            sl = pl.ds(i * 16, 16)
            zi[sl] = jnp.zeros((16,), jnp.int32)
            zf[sl] = jnp.zeros((16,), jnp.float32)
            return cy

        lax.fori_loop(0, 200, zr, 0)
        for i in range(8):
            z = pl.multiple_of(t * 25600 + i * 3200, 32)
            pltpu.sync_copy(zi, psp.at[pl.ds(z, 3200)])
            pltpu.sync_copy(zf, wsp.at[pl.ds(z, 3200)])
        zd = pl.multiple_of(t * 1568, 32)
        pltpu.sync_copy(zf.at[pl.ds(0, 1568)], dsp.at[pl.ds(zd, 1568)])
        plsc.subcore_barrier()

        def sc(g, cy):
            o = pl.multiple_of(ebase + g * 512, 8)
            lds = [pltpu.async_copy(pkd_h.at[pl.ds(o, 512)], vv, semL),
                   pltpu.async_copy(w_h.at[pl.ds(o, 512)], wv, semL)]
            for j in range(4):
                oj = pl.multiple_of(o + j * 128, 8)
                lds.append(pltpu.async_copy(pos_h.at[pl.ds(oj, 128)], pbufs[j], semL))
                lds.append(pltpu.async_copy(dloc_h.at[pl.ds(oj, 128)], dbufs[j], semL))
            for l in lds:
                l.wait()
            sts = []
            for j in range(4):
                sl = pl.ds(j * 128, 128)
                sts.append(pltpu.async_copy(vv.at[sl], psp.at[pbufs[j]], semS))
                sts.append(pltpu.async_copy(wv.at[sl], wsp.at[pbufs[j]], semS))
                sts.append(pltpu.async_copy(wv.at[sl], dsp.at[dbufs[j]], semS, add=True))
            for s in sts:
                s.wait()
            return cy

        lax.fori_loop(0, 49, sc, 0)
        plsc.subcore_barrier()
        for i in range(8):
            zs = pl.multiple_of(t * 25536 + i * 3192, 8)
            zh = pl.multiple_of(c * _HCAP + t * 25536 + i * 3192, 8)
            pltpu.sync_copy(psp.at[pl.ds(zs, 3192)], zi.at[pl.ds(0, 3192)])
            pltpu.sync_copy(zi.at[pl.ds(0, 3192)], opkd_h.at[pl.ds(zh, 3192)])
            pltpu.sync_copy(wsp.at[pl.ds(zs, 3192)], zf.at[pl.ds(0, 3192)])
            pltpu.sync_copy(zf.at[pl.ds(0, 3192)], ow_h.at[pl.ds(zh, 3192)])
        zd2 = pl.multiple_of(t * 1568, 32)
        pltpu.sync_copy(dsp.at[pl.ds(zd2, 1568)], zf.at[pl.ds(0, 1568)])
        zd3 = pl.multiple_of(c * _NUP + t * 1568, 32)
        pltpu.sync_copy(zf.at[pl.ds(0, 1568)], odeg_h.at[pl.ds(zd3, 1568)])

    return k(pos_f, pkd_f, dloc_f, w_f)


def _sc_norm(pkd_s, w_s, dinv, meta32):
    @functools.partial(
        pl.kernel,
        out_type=jax.ShapeDtypeStruct((_CAPX,), jnp.float32),
        mesh=_sc_mesh(),
        scratch_types=[
            pltpu.VMEM((2048,), jnp.int32),
            pltpu.VMEM((128,), jnp.int32),
            pltpu.VMEM((128,), jnp.int32),
            pltpu.VMEM((128,), jnp.float32),
            pltpu.VMEM((128,), jnp.float32),
            pltpu.VMEM((128,), jnp.float32),
        ],
    )
    def k(pkd_h, w_h, dinv_h, m_h, out_h, mv, pv, iv, av, bv, wv):
        c = lax.axis_index("c")
        t = lax.axis_index("s")
        pltpu.sync_copy(m_h, mv)

        def bucket(bl, cy):
            gb = c * 4 + bl
            nbase = gb * _BUK
            mb = mv[pl.ds(pl.multiple_of((bl * 32 + t * 2 + c) * 16, 16), 16)]
            off = mb[0]
            nch = mb[1]

            def chunk(kk, cy2):
                o = pl.multiple_of(off + kk * _CH, _CH)
                pltpu.sync_copy(pkd_h.at[pl.ds(o, _CH)], pv)
                for q in range(8):
                    sl = pl.ds(q * 16, 16)
                    iv[sl] = lax.shift_right_logical(pv[sl], 13)
                pltpu.sync_copy(dinv_h.at[iv], av)
                for q in range(8):
                    sl = pl.ds(q * 16, 16)
                    iv[sl] = (pv[sl] & 8191) + nbase
                pltpu.sync_copy(dinv_h.at[iv], bv)
                pltpu.sync_copy(w_h.at[pl.ds(o, _CH)], wv)
                for q in range(8):
                    sl = pl.ds(q * 16, 16)
                    wv[sl] = wv[sl] * av[sl] * bv[sl]
                pltpu.sync_copy(wv, out_h.at[pl.ds(o, _CH)])
                return cy2

            lax.fori_loop(0, nch, chunk, 0)
            return cy

        lax.fori_loop(0, 4, bucket, 0)

    return k(pkd_s, w_s, dinv, meta32)


def _sc_prop(x, accin, dinv2, pkd_s, norm_s, meta32):
    @functools.partial(
        pl.kernel,
        out_type=(jax.ShapeDtypeStruct((_NP, 128), jnp.float32),
                  jax.ShapeDtypeStruct((_NP, 128), jnp.float32)),
        mesh=_sc_mesh(),
        scratch_types=[
            pltpu.VMEM((2048,), jnp.int32),
            pltpu.VMEM((128,), jnp.int32),
            pltpu.VMEM((128,), jnp.int32),
            pltpu.VMEM((128,), jnp.float32),
            pltpu.VMEM((128,), jnp.float32),
            pltpu.VMEM((128,), jnp.int32),
            pltpu.VMEM((128,), jnp.int32),
            pltpu.VMEM((128,), jnp.int32),
            pltpu.VMEM((128,), jnp.int32),
            pltpu.VMEM((128, 128), jnp.float32),
            pltpu.VMEM((128, 128), jnp.float32),
            pltpu.VMEM((64, 128), jnp.float32),
            pltpu.VMEM((64, 128), jnp.float32),
            pltpu.VMEM((64, 128), jnp.float32),
            pltpu.VMEM((64,), jnp.float32),
            pltpu.VMEM_SHARED((_BUK, 128), jnp.float32),
            pltpu.SemaphoreType.DMA,
            pltpu.SemaphoreType.DMA,
            pltpu.SemaphoreType.DMA,
            pltpu.SemaphoreType.DMA,
        ],
    )
    def k(x_h, ai_h, d2_h, pkd_h, nrm_h, m_h, y_h, ao_h,
          mv, pk0, pk1, nv0, nv1, iv0, iv1, dl0, dl1, rw0, rw1,
          sb, xb, ab, dv64, asp, semG0, semG1, semS0, semS1):
        c = lax.axis_index("c")
        t = lax.axis_index("s")
        pltpu.sync_copy(m_h, mv)
        sets = ((pk0, nv0, iv0, dl0, rw0, semG0, semS0),
                (pk1, nv1, iv1, dl1, rw1, semG1, semS1))

        def load_and_fire(kk, off, s):
            pk, nv, iv, dl, rw, semG, _ = sets[s]
            o = pl.multiple_of(off + kk * _CH, _CH)
            pltpu.sync_copy(pkd_h.at[pl.ds(o, _CH)], pk)
            pltpu.sync_copy(nrm_h.at[pl.ds(o, _CH)], nv)
            for q in range(8):
                sl = pl.ds(q * 16, 16)
                p = pk[sl]
                iv[sl] = lax.shift_right_logical(p, 13)
                dl[sl] = p & 8191
            pltpu.async_copy(x_h.at[iv], rw, semG)

        def scale_and_scatter(s):
            _, nv, _, dl, rw, _, semS = sets[s]

            def sgrp(q, cy3):
                s16 = nv[pl.ds(q * 16, 16)]
                for l in range(16):
                    i = q * 16 + l
                    sc_ = s16[l]
                    for d in range(8):
                        sl = pl.ds(d * 16, 16)
                        rw[i, sl] = rw[i, sl] * sc_
                return cy3

            lax.fori_loop(0, 8, sgrp, 0)
            pltpu.async_copy(rw, asp.at[dl], semS, add=True)

        def wait_gather(s):
            _, _, iv, _, rw, semG, _ = sets[s]
            pltpu.make_async_copy(x_h.at[iv], rw, semG).wait()

        def wait_scatter(s):
            _, _, _, dl, rw, _, semS = sets[s]
            pltpu.make_async_copy(rw, asp.at[dl], semS).wait()

        def bucket(bl, cy):
            gb = c * 4 + bl
            nbase = gb * _BUK
            mb = mv[pl.ds(pl.multiple_of((bl * 32 + t * 2 + c) * 16, 16), 16)]
            off = mb[0]
            nch = mb[1]

            def zr(i, cy2):
                for d in range(8):
                    sb[i, pl.ds(d * 16, 16)] = jnp.zeros((16,), jnp.float32)
                return cy2

            lax.fori_loop(0, 64, zr, 0)

            def zc(i, cy2):
                pltpu.sync_copy(sb, asp.at[pl.ds(t * 392 + i * 64, 64)])
                return cy2

            lax.fori_loop(0, 6, zc, 0)
            pltpu.sync_copy(sb.at[pl.ds(0, 8)], asp.at[pl.ds(t * 392 + 384, 8)])
            plsc.subcore_barrier()

            def step(kk, cy2):
                o = pl.multiple_of(off + kk * _CH, _CH)
                pltpu.sync_copy(pkd_h.at[pl.ds(o, _CH)], pk0)
                pltpu.sync_copy(nrm_h.at[pl.ds(o, _CH)], nv0)
                for q in range(8):
                    sl = pl.ds(q * 16, 16)
                    p = pk0[sl]
                    iv0[sl] = lax.shift_right_logical(p, 13)
                    dl0[sl] = p & 8191
                pltpu.sync_copy(x_h.at[iv0], rw0)

                def sgrp(q, cy3):
                    s16 = nv0[pl.ds(q * 16, 16)]
                    for l in range(16):
                        i = q * 16 + l
                        sc_ = s16[l]
                        for d in range(8):
                            sl = pl.ds(d * 16, 16)
                            rw0[i, sl] = rw0[i, sl] * sc_
                    return cy3

                lax.fori_loop(0, 8, sgrp, 0)
                pltpu.sync_copy(rw0, asp.at[dl0], add=True)
                return cy2

            lax.fori_loop(0, nch, step, 0)
            plsc.subcore_barrier()

            nout = jnp.where(t < 2, 7, 6)

            def och(kk, cy2):
                j = t + kk * 16
                g0 = pl.multiple_of(nbase + j * 64, 64)
                pltpu.sync_copy(asp.at[pl.ds(j * 64, 64)], sb)
                pltpu.sync_copy(x_h.at[pl.ds(g0, 64)], xb)
                pltpu.sync_copy(ai_h.at[pl.ds(g0, 64)], ab)
                pltpu.sync_copy(d2_h.at[pl.ds(g0, 64)], dv64)

                def ogrp(q, cy3):
                    s16 = dv64[pl.ds(q * 16, 16)]
                    for l in range(16):
                        i = q * 16 + l
                        sc_ = s16[l]
                        for d in range(8):
                            sl = pl.ds(d * 16, 16)
                            y = sb[i, sl] + xb[i, sl] * sc_
                            sb[i, sl] = y
                            ab[i, sl] = ab[i, sl] + y
                    return cy3

                lax.fori_loop(0, 4, ogrp, 0)
                pltpu.sync_copy(sb, y_h.at[pl.ds(g0, 64)])
                pltpu.sync_copy(ab, ao_h.at[pl.ds(g0, 64)])
                return cy2

            lax.fori_loop(0, nout, och, 0)
            plsc.subcore_barrier()
            return cy

        lax.fori_loop(0, 4, bucket, 0)

    return k(x, accin, dinv2, pkd_s, norm_s, meta32)


def kernel(user_emb, artist_emb, album_emb, item_audio_emb, edge_attr,
           W1, b1, W2, b2, Wp, bp, edge_src, edge_dst, artist_ids, album_ids):
    f32 = jnp.float32
    i32 = jnp.int32

    wrow = _tc_edge_mlp(edge_attr.T, W1.T, b1.reshape(32, 1), W2.T, b2.reshape(1, 1))
    w = wrow[0]

    dst_off = edge_dst + _NUP
    src_h = jnp.concatenate([dst_off, edge_src])
    dst_h = jnp.concatenate([edge_src, dst_off])
    w_h = jnp.concatenate([w, w])
    key = dst_h // _BUK
    onehot = (key[:, None] == jnp.arange(_NBUK, dtype=key.dtype)[None, :]).astype(i32)
    ranks = jnp.cumsum(onehot, axis=0)
    rank = jnp.take_along_axis(ranks, key[:, None], axis=1)[:, 0] - 1
    cnt = ranks[-1]
    per_tile = (cnt + 15) // 16
    pt_e = per_tile[key]
    t_idx = rank // pt_e
    r_in_t = rank - t_idx * pt_e
    seg_cnt = jnp.clip(cnt[:, None] - jnp.arange(16, dtype=i32)[None, :] * per_tile[:, None],
                       0, per_tile[:, None])
    cnt_f = seg_cnt.reshape(-1).astype(i32)
    scf = ((cnt_f + _CH - 1) // _CH) * _CH
    cume = jnp.concatenate([jnp.zeros((1,), i32), jnp.cumsum(scf)[:-1].astype(i32)])
    half_of_seg = (jnp.arange(_NSEG, dtype=i32) >= 64).astype(i32)
    local_off = cume - half_of_seg * cume[64]
    seg_off = local_off + half_of_seg * _HCAP
    segnch = scf // _CH
    seg_e = key * 16 + t_idx
    pos_loc = local_off[seg_e] + r_in_t
    pkd = src_h * 8192 + (dst_h - key * _BUK)
    dloc = jnp.concatenate([dst_h[:_E], dst_h[_E:] - _NUP])
    padp = _HCAP + (jnp.arange(_EHS - _E, dtype=i32) % _CH)
    zi_p = jnp.zeros((_EHS - _E,), i32)
    zf_p = jnp.zeros((_EHS - _E,), f32)
    pos_f = jnp.concatenate([pos_loc[:_E], padp, pos_loc[_E:], padp])
    pkd_f = jnp.concatenate([pkd[:_E], zi_p, pkd[_E:], zi_p])
    dloc_f = jnp.concatenate([dloc[:_E], zi_p, dloc[_E:], zi_p])
    w_f = jnp.concatenate([w_h[:_E], zf_p, w_h[_E:], zf_p])

    tw = jnp.arange(32, dtype=i32) // 2
    cw = jnp.arange(32, dtype=i32) % 2
    rows_m = []
    for bl in range(4):
        segb = (cw * 4 + bl) * 16 + tw
        rows_m.append(jnp.stack([seg_off[segb], segnch[segb]] +
                                [jnp.zeros((32,), i32)] * 14, axis=1))
    meta32 = jnp.concatenate(rows_m, axis=0).reshape(-1)

    pkd_s, w_s, degp = _sc_build(pos_f, pkd_f, dloc_f, w_f)
    deg = degp + 1.0
    dinv = deg ** -0.5
    dinv2 = 1.0 / deg

    norm_s = _sc_norm(pkd_s, w_s, dinv, meta32)

    aid_p = jnp.zeros((_MIDP,), i32).at[:_NI].set(artist_ids)
    bid_p = jnp.zeros((_MIDP,), i32).at[:_NI].set(album_ids)
    meta_p = _sc_meta(artist_emb, album_emb, aid_p, bid_p)

    user_x = _pl_normalize(user_emb, 1000)
    item_x = _tc_item(item_audio_emb, meta_p[:_NI], Wp[:128], Wp[128:], bp.reshape(1, 128))
    zpad = jnp.zeros((_NUP - _NU, 128), f32)
    x = jnp.concatenate([user_x, zpad, item_x, zpad], axis=0)

    acc = x
    for _ in range(_NL):
        x, acc = _sc_prop(x, acc, dinv2, pkd_s, norm_s, meta32)

    out = _pl_normalize(acc * 0.25, 784)
    return out[:_NU], out[_NUP:_NUP + _NI]


# speedup vs baseline: 3.0326x; 1.0642x over previous
import functools

import jax
import jax.numpy as jnp
from jax import lax
from jax.experimental import pallas as pl
from jax.experimental.pallas import tpu as pltpu
from jax.experimental.pallas import tpu_sc as plsc

_NU = 25000
_NI = 25000
_NN = 50000
_E = 400000
_E2 = 800000
_NL = 3

_BUK = 6272
_NBUK = 8
_NP = _NBUK * _BUK
_NSEG = _NBUK * 16
_CH = 128
_NUP = 25088
_HCAP = 408576
_HS = 409600
_CAPX = 2 * _HCAP
_EHS = 401408
_MIDP = 25088


def _norm_body(x_ref, o_ref):
    x = x_ref[...]
    n = jnp.sqrt(jnp.sum(x * x, axis=-1, keepdims=True))
    o_ref[...] = x / jnp.maximum(n, 1e-12)


def _pl_normalize(x, blk):
    m, d = x.shape
    return pl.pallas_call(
        _norm_body,
        out_shape=jax.ShapeDtypeStruct((m, d), x.dtype),
        grid=(m // blk,),
        in_specs=[pl.BlockSpec((blk, d), lambda i: (i, 0))],
        out_specs=pl.BlockSpec((blk, d), lambda i: (i, 0)),
    )(x)


def _tc_edge_mlp(attrT, W1T, b1c, W2T, b2c):
    def body(a_ref, w1_ref, b1_ref, w2_ref, b2_ref, o_ref):
        a = a_ref[...]
        h = jnp.maximum(
            jnp.dot(w1_ref[...], a, preferred_element_type=jnp.float32) + b1_ref[...], 0.0)
        w = jax.nn.sigmoid(
            jnp.dot(w2_ref[...], h, preferred_element_type=jnp.float32) + b2_ref[...])
        o_ref[...] = jnp.maximum(w, 1e-6)

    return pl.pallas_call(
        body,
        out_shape=jax.ShapeDtypeStruct((1, _E), jnp.float32),
        grid=(125,),
        in_specs=[
            pl.BlockSpec((4, 3200), lambda i: (0, i)),
            pl.BlockSpec((32, 4), lambda i: (0, 0)),
            pl.BlockSpec((32, 1), lambda i: (0, 0)),
            pl.BlockSpec((1, 32), lambda i: (0, 0)),
            pl.BlockSpec((1, 1), lambda i: (0, 0)),
        ],
        out_specs=pl.BlockSpec((1, 3200), lambda i: (0, i)),
    )(attrT, W1T, b1c, W2T, b2c)


def _tc_item(audio, meta_p, Wpa, Wpm, bpr):
    def body(au, me, wa, wm, bp_, o):
        pre = (jnp.dot(au[...], wa[...], preferred_element_type=jnp.float32)
               + jnp.dot(me[...], wm[...], preferred_element_type=jnp.float32)
               + bp_[...])
        n = jnp.sqrt(jnp.sum(pre * pre, axis=1, keepdims=True))
        o[...] = pre / jnp.maximum(n, 1e-12)

    return pl.pallas_call(
        body,
        out_shape=jax.ShapeDtypeStruct((_NI, 128), jnp.float32),
        grid=(25,),
        in_specs=[
            pl.BlockSpec((1000, 128), lambda i: (i, 0)),
            pl.BlockSpec((1000, 128), lambda i: (i, 0)),
            pl.BlockSpec((128, 128), lambda i: (0, 0)),
            pl.BlockSpec((128, 128), lambda i: (0, 0)),
            pl.BlockSpec((1, 128), lambda i: (0, 0)),
        ],
        out_specs=pl.BlockSpec((1000, 128), lambda i: (i, 0)),
    )(audio, meta_p, Wpa, Wpm, bpr)


def _sc_mesh():
    return plsc.VectorSubcoreMesh(core_axis_name="c", subcore_axis_name="s")


def _sc_meta(artist_emb, album_emb, aid_p, bid_p):
    @functools.partial(
        pl.kernel,
        out_type=jax.ShapeDtypeStruct((_MIDP, 128), jnp.float32),
        mesh=_sc_mesh(),
        scratch_types=[
            pltpu.VMEM((112,), jnp.int32),
            pltpu.VMEM((112, 128), jnp.float32),
            pltpu.VMEM((112, 128), jnp.float32),
        ],
    )
    def k(art_h, alb_h, aid_h, bid_h, out_h, idx_v, ra, rb):
        c = lax.axis_index("c")
        t = lax.axis_index("s")
        base = (t * 2 + c) * 784

        def chunk(kk, carry):
            r0 = pl.multiple_of(base + kk * 112, 16)
            pltpu.sync_copy(aid_h.at[pl.ds(r0, 112)], idx_v)
            pltpu.sync_copy(art_h.at[idx_v], ra)
            pltpu.sync_copy(bid_h.at[pl.ds(r0, 112)], idx_v)
            pltpu.sync_copy(alb_h.at[idx_v], rb)

            def row(i, cy):
                for d in range(8):
                    sl = pl.ds(d * 16, 16)
                    ra[i, sl] = ra[i, sl] + rb[i, sl]
                return cy

            lax.fori_loop(0, 112, row, 0)
            pltpu.sync_copy(ra, out_h.at[pl.ds(r0, 112)])
            return carry

        lax.fori_loop(0, 7, chunk, 0)

    return k(artist_emb, album_emb, aid_p, bid_p)


def _sc_build(pos_f, pkd_f, dloc_f, w_f):
    @functools.partial(
        pl.kernel,
        out_type=(jax.ShapeDtypeStruct((_CAPX,), jnp.int32),
                  jax.ShapeDtypeStruct((_CAPX,), jnp.float32),
                  jax.ShapeDtypeStruct((2 * _NUP,), jnp.float32)),
        mesh=_sc_mesh(),
        scratch_types=[
            pltpu.VMEM((512,), jnp.int32),
            pltpu.VMEM((512,), jnp.float32),
            pltpu.VMEM((128,), jnp.int32),
            pltpu.VMEM((128,), jnp.int32),
            pltpu.VMEM((128,), jnp.int32),
            pltpu.VMEM((128,), jnp.int32),
            pltpu.VMEM((128,), jnp.int32),
            pltpu.VMEM((128,), jnp.int32),
            pltpu.VMEM((128,), jnp.int32),
            pltpu.VMEM((128,), jnp.int32),
            pltpu.VMEM((3200,), jnp.int32),
            pltpu.VMEM((3200,), jnp.float32),
            pltpu.VMEM_SHARED((_HS,), jnp.int32),
            pltpu.VMEM_SHARED((_HS,), jnp.float32),
            pltpu.VMEM_SHARED((_NUP,), jnp.float32),
            pltpu.SemaphoreType.DMA,
            pltpu.SemaphoreType.DMA,
        ],
    )
    def k(pos_h, pkd_h, dloc_h, w_h, opkd_h, ow_h, odeg_h,
          vv, wv, p0, p1, p2, p3, d0, d1, d2, d3, zi, zf,
          psp, wsp, dsp, semL, semS):
        c = lax.axis_index("c")
        t = lax.axis_index("s")
        ebase = c * _EHS + t * 25088
        pbufs = [p0, p1, p2, p3]
        dbufs = [d0, d1, d2, d3]

        def zr(i, cy):
            sl = pl.ds(i * 16, 16)
            zi[sl] = jnp.zeros((16,), jnp.int32)
            zf[sl] = jnp.zeros((16,), jnp.float32)
            return cy

        lax.fori_loop(0, 200, zr, 0)
        for i in range(8):
            z = pl.multiple_of(t * 25600 + i * 3200, 32)
            pltpu.sync_copy(zi, psp.at[pl.ds(z, 3200)])
            pltpu.sync_copy(zf, wsp.at[pl.ds(z, 3200)])
        zd = pl.multiple_of(t * 1568, 32)
        pltpu.sync_copy(zf.at[pl.ds(0, 1568)], dsp.at[pl.ds(zd, 1568)])
        plsc.subcore_barrier()

        def sc(g, cy):
            o = pl.multiple_of(ebase + g * 512, 8)
            lds = [pltpu.async_copy(pkd_h.at[pl.ds(o, 512)], vv, semL),
                   pltpu.async_copy(w_h.at[pl.ds(o, 512)], wv, semL)]
            for j in range(4):
                oj = pl.multiple_of(o + j * 128, 8)
                lds.append(pltpu.async_copy(pos_h.at[pl.ds(oj, 128)], pbufs[j], semL))
                lds.append(pltpu.async_copy(dloc_h.at[pl.ds(oj, 128)], dbufs[j], semL))
            for l in lds:
                l.wait()
            sts = []
            for j in range(4):
                sl = pl.ds(j * 128, 128)
                sts.append(pltpu.async_copy(vv.at[sl], psp.at[pbufs[j]], semS))
                sts.append(pltpu.async_copy(wv.at[sl], wsp.at[pbufs[j]], semS))
                sts.append(pltpu.async_copy(wv.at[sl], dsp.at[dbufs[j]], semS, add=True))
            for s in sts:
                s.wait()
            return cy

        lax.fori_loop(0, 49, sc, 0)
        plsc.subcore_barrier()
        for i in range(8):
            zs = pl.multiple_of(t * 25536 + i * 3192, 8)
            zh = pl.multiple_of(c * _HCAP + t * 25536 + i * 3192, 8)
            pltpu.sync_copy(psp.at[pl.ds(zs, 3192)], zi.at[pl.ds(0, 3192)])
            pltpu.sync_copy(zi.at[pl.ds(0, 3192)], opkd_h.at[pl.ds(zh, 3192)])
            pltpu.sync_copy(wsp.at[pl.ds(zs, 3192)], zf.at[pl.ds(0, 3192)])
            pltpu.sync_copy(zf.at[pl.ds(0, 3192)], ow_h.at[pl.ds(zh, 3192)])
        zd2 = pl.multiple_of(t * 1568, 32)
        pltpu.sync_copy(dsp.at[pl.ds(zd2, 1568)], zf.at[pl.ds(0, 1568)])
        zd3 = pl.multiple_of(c * _NUP + t * 1568, 32)
        pltpu.sync_copy(zf.at[pl.ds(0, 1568)], odeg_h.at[pl.ds(zd3, 1568)])

    return k(pos_f, pkd_f, dloc_f, w_f)


def _sc_norm(pkd_s, w_s, dinv, meta32):
    @functools.partial(
        pl.kernel,
        out_type=jax.ShapeDtypeStruct((_CAPX,), jnp.float32),
        mesh=_sc_mesh(),
        scratch_types=[
            pltpu.VMEM((2048,), jnp.int32),
            pltpu.VMEM((128,), jnp.int32),
            pltpu.VMEM((128,), jnp.int32),
            pltpu.VMEM((128,), jnp.float32),
            pltpu.VMEM((128,), jnp.float32),
            pltpu.VMEM((128,), jnp.float32),
        ],
    )
    def k(pkd_h, w_h, dinv_h, m_h, out_h, mv, pv, iv, av, bv, wv):
        c = lax.axis_index("c")
        t = lax.axis_index("s")
        pltpu.sync_copy(m_h, mv)

        def bucket(bl, cy):
            gb = c * 4 + bl
            nbase = gb * _BUK
            mb = mv[pl.ds(pl.multiple_of((bl * 32 + t * 2 + c) * 16, 16), 16)]
            off = mb[0]
            nch = mb[1]

            def chunk(kk, cy2):
                o = pl.multiple_of(off + kk * _CH, _CH)
                pltpu.sync_copy(pkd_h.at[pl.ds(o, _CH)], pv)
                for q in range(8):
                    sl = pl.ds(q * 16, 16)
                    iv[sl] = lax.shift_right_logical(pv[sl], 13)
                pltpu.sync_copy(dinv_h.at[iv], av)
                for q in range(8):
                    sl = pl.ds(q * 16, 16)
                    iv[sl] = (pv[sl] & 8191) + nbase
                pltpu.sync_copy(dinv_h.at[iv], bv)
                pltpu.sync_copy(w_h.at[pl.ds(o, _CH)], wv)
                for q in range(8):
                    sl = pl.ds(q * 16, 16)
                    wv[sl] = wv[sl] * av[sl] * bv[sl]
                pltpu.sync_copy(wv, out_h.at[pl.ds(o, _CH)])
                return cy2

            lax.fori_loop(0, nch, chunk, 0)
            return cy

        lax.fori_loop(0, 4, bucket, 0)

    return k(pkd_s, w_s, dinv, meta32)


def _sc_prop(x, accin, dinv2, pkd_s, norm_s, meta32):
    @functools.partial(
        pl.kernel,
        out_type=(jax.ShapeDtypeStruct((_NP, 128), jnp.float32),
                  jax.ShapeDtypeStruct((_NP, 128), jnp.float32)),
        mesh=_sc_mesh(),
        scratch_types=[
            pltpu.VMEM((2048,), jnp.int32),
            pltpu.VMEM((128,), jnp.int32),
            pltpu.VMEM((128,), jnp.int32),
            pltpu.VMEM((128,), jnp.float32),
            pltpu.VMEM((128,), jnp.float32),
            pltpu.VMEM((128,), jnp.int32),
            pltpu.VMEM((128,), jnp.int32),
            pltpu.VMEM((128,), jnp.int32),
            pltpu.VMEM((128,), jnp.int32),
            pltpu.VMEM((128, 128), jnp.float32),
            pltpu.VMEM((128, 128), jnp.float32),
            pltpu.VMEM((64, 128), jnp.float32),
            pltpu.VMEM((64, 128), jnp.float32),
            pltpu.VMEM((64, 128), jnp.float32),
            pltpu.VMEM((64,), jnp.float32),
            pltpu.VMEM_SHARED((_BUK, 128), jnp.float32),
            pltpu.SemaphoreType.DMA,
            pltpu.SemaphoreType.DMA,
            pltpu.SemaphoreType.DMA,
            pltpu.SemaphoreType.DMA,
        ],
    )
    def k(x_h, ai_h, d2_h, pkd_h, nrm_h, m_h, y_h, ao_h,
          mv, pk0, pk1, nv0, nv1, iv0, iv1, dl0, dl1, rw0, rw1,
          sb, xb, ab, dv64, asp, semG0, semG1, semS0, semS1):
        c = lax.axis_index("c")
        t = lax.axis_index("s")
        pltpu.sync_copy(m_h, mv)
        sets = ((pk0, nv0, iv0, dl0, rw0, semG0, semS0),
                (pk1, nv1, iv1, dl1, rw1, semG1, semS1))

        def load_and_fire(kk, off, s):
            pk, nv, iv, dl, rw, semG, _ = sets[s]
            o = pl.multiple_of(off + kk * _CH, _CH)
            pltpu.sync_copy(pkd_h.at[pl.ds(o, _CH)], pk)
            pltpu.sync_copy(nrm_h.at[pl.ds(o, _CH)], nv)
            for q in range(8):
                sl = pl.ds(q * 16, 16)
                p = pk[sl]
                iv[sl] = lax.shift_right_logical(p, 13)
                dl[sl] = p & 8191
            pltpu.async_copy(x_h.at[iv], rw, semG)

        def scale_and_scatter(s):
            _, nv, _, dl, rw, _, semS = sets[s]

            def sgrp(q, cy3):
                s16 = nv[pl.ds(q * 16, 16)]
                for l in range(16):
                    i = q * 16 + l
                    sc_ = s16[l]
                    for d in range(8):
                        sl = pl.ds(d * 16, 16)
                        rw[i, sl] = rw[i, sl] * sc_
                return cy3

            lax.fori_loop(0, 8, sgrp, 0)
            pltpu.async_copy(rw, asp.at[dl], semS, add=True)

        def wait_gather(s):
            _, _, iv, _, rw, semG, _ = sets[s]
            pltpu.make_async_copy(x_h.at[iv], rw, semG).wait()

        def wait_scatter(s):
            _, _, _, dl, rw, _, semS = sets[s]
            pltpu.make_async_copy(rw, asp.at[dl], semS).wait()

        def bucket(bl, cy):
            gb = c * 4 + bl
            nbase = gb * _BUK
            mb = mv[pl.ds(pl.multiple_of((bl * 32 + t * 2 + c) * 16, 16), 16)]
            off = mb[0]
            nch = mb[1]

            def zr(i, cy2):
                for d in range(8):
                    sb[i, pl.ds(d * 16, 16)] = jnp.zeros((16,), jnp.float32)
                return cy2

            lax.fori_loop(0, 64, zr, 0)

            def zc(i, cy2):
                pltpu.sync_copy(sb, asp.at[pl.ds(t * 392 + i * 64, 64)])
                return cy2

            lax.fori_loop(0, 6, zc, 0)
            pltpu.sync_copy(sb.at[pl.ds(0, 8)], asp.at[pl.ds(t * 392 + 384, 8)])
            plsc.subcore_barrier()

            @pl.when(nch > 0)
            def _():
                load_and_fire(0, off, 0)

            def step(kk, cy2):
                def work(sA, sB):
                    wait_gather(sA)

                    @pl.when(kk > 0)
                    def _():
                        wait_scatter(sB)

                    @pl.when(kk + 1 < nch)
                    def _():
                        load_and_fire(kk + 1, off, sB)

                    scale_and_scatter(sA)

                @pl.when(kk % 2 == 0)
                def _():
                    work(0, 1)

                @pl.when(kk % 2 == 1)
                def _():
                    work(1, 0)

                return cy2

            lax.fori_loop(0, nch, step, 0)

            @pl.when(nch > 0)
            def _():
                @pl.when((nch - 1) % 2 == 0)
                def _():
                    wait_scatter(0)

                @pl.when((nch - 1) % 2 == 1)
                def _():
                    wait_scatter(1)

            plsc.subcore_barrier()

            nout = jnp.where(t < 2, 7, 6)

            def och(kk, cy2):
                j = t + kk * 16
                g0 = pl.multiple_of(nbase + j * 64, 64)
                pltpu.sync_copy(asp.at[pl.ds(j * 64, 64)], sb)
                pltpu.sync_copy(x_h.at[pl.ds(g0, 64)], xb)
                pltpu.sync_copy(ai_h.at[pl.ds(g0, 64)], ab)
                pltpu.sync_copy(d2_h.at[pl.ds(g0, 64)], dv64)

                def ogrp(q, cy3):
                    s16 = dv64[pl.ds(q * 16, 16)]
                    for l in range(16):
                        i = q * 16 + l
                        sc_ = s16[l]
                        for d in range(8):
                            sl = pl.ds(d * 16, 16)
                            y = sb[i, sl] + xb[i, sl] * sc_
                            sb[i, sl] = y
                            ab[i, sl] = ab[i, sl] + y
                    return cy3

                lax.fori_loop(0, 4, ogrp, 0)
                pltpu.sync_copy(sb, y_h.at[pl.ds(g0, 64)])
                pltpu.sync_copy(ab, ao_h.at[pl.ds(g0, 64)])
                return cy2

            lax.fori_loop(0, nout, och, 0)
            plsc.subcore_barrier()
            return cy

        lax.fori_loop(0, 4, bucket, 0)

    return k(x, accin, dinv2, pkd_s, norm_s, meta32)


def kernel(user_emb, artist_emb, album_emb, item_audio_emb, edge_attr,
           W1, b1, W2, b2, Wp, bp, edge_src, edge_dst, artist_ids, album_ids):
    f32 = jnp.float32
    i32 = jnp.int32

    wrow = _tc_edge_mlp(edge_attr.T, W1.T, b1.reshape(32, 1), W2.T, b2.reshape(1, 1))
    w = wrow[0]

    dst_off = edge_dst + _NUP
    src_h = jnp.concatenate([dst_off, edge_src])
    dst_h = jnp.concatenate([edge_src, dst_off])
    w_h = jnp.concatenate([w, w])
    key = dst_h // _BUK
    onehot = (key[:, None] == jnp.arange(_NBUK, dtype=key.dtype)[None, :]).astype(i32)
    ranks = jnp.cumsum(onehot, axis=0)
    rank = jnp.take_along_axis(ranks, key[:, None], axis=1)[:, 0] - 1
    cnt = ranks[-1]
    per_tile = (cnt + 15) // 16
    pt_e = per_tile[key]
    t_idx = rank // pt_e
    r_in_t = rank - t_idx * pt_e
    seg_cnt = jnp.clip(cnt[:, None] - jnp.arange(16, dtype=i32)[None, :] * per_tile[:, None],
                       0, per_tile[:, None])
    cnt_f = seg_cnt.reshape(-1).astype(i32)
    scf = ((cnt_f + _CH - 1) // _CH) * _CH
    cume = jnp.concatenate([jnp.zeros((1,), i32), jnp.cumsum(scf)[:-1].astype(i32)])
    half_of_seg = (jnp.arange(_NSEG, dtype=i32) >= 64).astype(i32)
    local_off = cume - half_of_seg * cume[64]
    seg_off = local_off + half_of_seg * _HCAP
    segnch = scf // _CH
    seg_e = key * 16 + t_idx
    pos_loc = local_off[seg_e] + r_in_t
    pkd = src_h * 8192 + (dst_h - key * _BUK)
    dloc = jnp.concatenate([dst_h[:_E], dst_h[_E:] - _NUP])
    padp = _HCAP + (jnp.arange(_EHS - _E, dtype=i32) % _CH)
    zi_p = jnp.zeros((_EHS - _E,), i32)
    zf_p = jnp.zeros((_EHS - _E,), f32)
    pos_f = jnp.concatenate([pos_loc[:_E], padp, pos_loc[_E:], padp])
    pkd_f = jnp.concatenate([pkd[:_E], zi_p, pkd[_E:], zi_p])
    dloc_f = jnp.concatenate([dloc[:_E], zi_p, dloc[_E:], zi_p])
    w_f = jnp.concatenate([w_h[:_E], zf_p, w_h[_E:], zf_p])

    tw = jnp.arange(32, dtype=i32) // 2
    cw = jnp.arange(32, dtype=i32) % 2
    rows_m = []
    for bl in range(4):
        segb = (cw * 4 + bl) * 16 + tw
        rows_m.append(jnp.stack([seg_off[segb], segnch[segb]] +
                                [jnp.zeros((32,), i32)] * 14, axis=1))
    meta32 = jnp.concatenate(rows_m, axis=0).reshape(-1)

    pkd_s, w_s, degp = _sc_build(pos_f, pkd_f, dloc_f, w_f)
    deg = degp + 1.0
    dinv = deg ** -0.5
    dinv2 = 1.0 / deg

    norm_s = _sc_norm(pkd_s, w_s, dinv, meta32)

    aid_p = jnp.zeros((_MIDP,), i32).at[:_NI].set(artist_ids)
    bid_p = jnp.zeros((_MIDP,), i32).at[:_NI].set(album_ids)
    meta_p = _sc_meta(artist_emb, album_emb, aid_p, bid_p)

    user_x = _pl_normalize(user_emb, 1000)
    item_x = _tc_item(item_audio_emb, meta_p[:_NI], Wp[:128], Wp[128:], bp.reshape(1, 128))
    zpad = jnp.zeros((_NUP - _NU, 128), f32)
    x = jnp.concatenate([user_x, zpad, item_x, zpad], axis=0)

    acc = x
    for _ in range(_NL):
        x, acc = _sc_prop(x, acc, dinv2, pkd_s, norm_s, meta32)

    out = _pl_normalize(acc * 0.25, 784)
    return out[:_NU], out[_NUP:_NUP + _NI]
